# baseline (speedup 1.0000x reference)
import functools

import jax
import jax.numpy as jnp
from jax import lax
from jax.experimental import pallas as pl
from jax.experimental.pallas import tpu as pltpu
from jax.experimental.pallas import tpu_sc as plsc

_TEMP = 0.05
_EPS = 1e-12
_LOG2E = 1.4426950408889634


def _sc_target_cos(table, idx, x, B, D):
    info = plsc.get_sparse_core_info()
    nw = info.num_cores * info.num_subcores
    bw = B // nw
    ng = bw // 16
    mesh = plsc.VectorSubcoreMesh(core_axis_name="c", subcore_axis_name="s")

    @functools.partial(
        pl.kernel,
        mesh=mesh,
        out_type=jax.ShapeDtypeStruct((B,), jnp.float32),
        compiler_params=pltpu.CompilerParams(
            use_tc_tiling_on_sc=False, needs_layout_passes=False),
        scratch_types=[
            pltpu.VMEM((bw,), jnp.int32),
            pltpu.VMEM((bw, D), jnp.float32),
            pltpu.VMEM((bw, D), jnp.float32),
            pltpu.VMEM((bw,), jnp.float32),
            pltpu.SemaphoreType.DMA,
        ],
    )
    def cos_kernel(table_hbm, idx_hbm, x_hbm, out_hbm, idx_v, rows_v, x_v,
                   out_v, sem):
        wid = lax.axis_index("s") * info.num_cores + lax.axis_index("c")
        base = wid * bw
        pltpu.sync_copy(idx_hbm.at[pl.ds(base, bw)], idx_v)
        pltpu.sync_copy(x_hbm.at[pl.ds(base, bw)], x_v)
        pltpu.async_copy(table_hbm.at[idx_v], rows_v, sem).wait()
        lane = jnp.arange(16, dtype=jnp.int32)
        for g in range(ng):
            row_ids = lane + (16 * g)
            d = jnp.zeros((16,), jnp.float32)
            s = jnp.zeros((16,), jnp.float32)
            for j in range(D):
                col = jnp.full((16,), j, jnp.int32)
                fv = plsc.load_gather(rows_v, [row_ids, col])
                xv = plsc.load_gather(x_v, [row_ids, col])
                d = d + fv * xv
                s = s + xv * xv
            i = plsc.bitcast(s, jnp.int32)
            y = plsc.bitcast(jnp.int32(0x5F3759DF) - (i >> 1), jnp.float32)
            for _ in range(3):
                y = y * (1.5 - 0.5 * s * y * y)
            denom = s * y + _EPS
            out_v[pl.ds(16 * g, 16)] = d / denom
        pltpu.sync_copy(out_v, out_hbm.at[pl.ds(base, bw)])

    return cos_kernel(table, idx, x)


def _tc_body(inv_b, x_ref, feat_ref, out_ref):
    i = pl.program_id(0)
    x = x_ref[...]
    norm = jnp.sqrt(jnp.sum(x * x, axis=1, keepdims=True))
    xn = x * (_LOG2E / _TEMP) / (norm + _EPS)
    l2 = jnp.dot(xn.astype(jnp.bfloat16), feat_ref[...],
                 preferred_element_type=jnp.float32)
    se = jnp.sum(jnp.exp2(l2), axis=1, keepdims=True)
    lse = jnp.log(se)
    partial = jnp.sum(lse, keepdims=True) * inv_b

    @pl.when(i == 0)
    def _init():
        out_ref[...] = jnp.zeros_like(out_ref)

    out_ref[...] += partial


def _tc_lse_mean(x, feat_t, BT):
    B, D = x.shape
    K = feat_t.shape[1]
    return pl.pallas_call(
        functools.partial(_tc_body, 1.0 / B),
        grid=(B // BT,),
        in_specs=[
            pl.BlockSpec((BT, D), lambda i: (i, 0)),
            pl.BlockSpec((D, K), lambda i: (0, 0)),
        ],
        out_specs=pl.BlockSpec((1, 1), lambda i: (0, 0)),
        out_shape=jax.ShapeDtypeStruct((1, 1), jnp.float32),
    )(x, feat_t)


def _comb_body(scale, lse_ref, t_ref, out_ref):
    out_ref[...] = lse_ref[...] - jnp.sum(t_ref[...], keepdims=True) * scale


def _combine(lse_sum, t_row):
    B = t_row.shape[1]
    return pl.pallas_call(
        functools.partial(_comb_body, 1.0 / (B * _TEMP)),
        out_shape=jax.ShapeDtypeStruct((1, 1), jnp.float32),
    )(lse_sum, t_row)


def kernel(inputs, targets, features):
    B, D = inputs.shape
    t = _sc_target_cos(features, targets.astype(jnp.int32), inputs, B, D)
    lse_sum = _tc_lse_mean(inputs, features.T.astype(jnp.bfloat16), 512)
    loss = _combine(lse_sum, t.reshape(1, B))
    return loss[0, 0]


# speedup vs baseline: 1.0037x; 1.0037x over previous
import functools

import jax
import jax.numpy as jnp
from jax import lax
from jax.experimental import pallas as pl
from jax.experimental.pallas import tpu as pltpu
from jax.experimental.pallas import tpu_sc as plsc

_TEMP = 0.05
_EPS = 1e-12
_LOG2E = 1.4426950408889634


def _sc_target_partials(table_pad, idx, x, B, D):
    info = plsc.get_sparse_core_info()
    nw = info.num_cores * info.num_subcores
    bw = B // nw
    ng = bw // 16
    dpad = table_pad.shape[1]
    mesh = plsc.VectorSubcoreMesh(core_axis_name="c", subcore_axis_name="s")

    @functools.partial(
        pl.kernel,
        mesh=mesh,
        out_type=jax.ShapeDtypeStruct((nw, 16), jnp.float32),
        compiler_params=pltpu.CompilerParams(needs_layout_passes=False),
        scratch_types=[
            pltpu.VMEM((bw,), jnp.int32),
            pltpu.VMEM((bw, dpad), jnp.float32),
            pltpu.VMEM((bw, D), jnp.float32),
            pltpu.VMEM((16,), jnp.float32),
            pltpu.SemaphoreType.DMA,
        ],
    )
    def cos_kernel(table_hbm, idx_hbm, x_hbm, out_hbm, idx_v, rows_v, x_v,
                   p_v, sem):
        wid = lax.axis_index("s") * info.num_cores + lax.axis_index("c")
        base = wid * bw
        pltpu.sync_copy(idx_hbm.at[pl.ds(base, bw)], idx_v)
        pltpu.sync_copy(x_hbm.at[pl.ds(base, bw)], x_v)
        pltpu.async_copy(table_hbm.at[idx_v], rows_v, sem).wait()
        lane = jnp.arange(16, dtype=jnp.int32)
        p = jnp.zeros((16,), jnp.float32)
        for g in range(ng):
            row_ids = lane + (16 * g)
            d = jnp.zeros((16,), jnp.float32)
            s = jnp.zeros((16,), jnp.float32)
            for j in range(D):
                col = jnp.full((16,), j, jnp.int32)
                fv = plsc.load_gather(rows_v, [row_ids, col])
                xv = plsc.load_gather(x_v, [row_ids, col])
                d = d + fv * xv
                s = s + xv * xv
            i = plsc.bitcast(s, jnp.int32)
            y = plsc.bitcast(jnp.int32(0x5F3759DF) - (i >> 1), jnp.float32)
            for _ in range(3):
                y = y * (1.5 - 0.5 * s * y * y)
            p = p + d / (s * y + _EPS)
        p_v[...] = p
        pltpu.sync_copy(p_v, out_hbm.at[wid])

    return cos_kernel(table_pad, idx, x)


def _tc_body(inv_b, x_ref, feat_ref, out_ref):
    i = pl.program_id(0)
    x = x_ref[...]
    norm = jnp.sqrt(jnp.sum(x * x, axis=1, keepdims=True))
    xn = x * (_LOG2E / _TEMP) / (norm + _EPS)
    l2 = jnp.dot(xn.astype(jnp.bfloat16), feat_ref[...],
                 preferred_element_type=jnp.float32)
    se = jnp.sum(jnp.exp2(l2), axis=1, keepdims=True)
    lse = jnp.log(se)
    partial = jnp.sum(lse, keepdims=True) * inv_b

    @pl.when(i == 0)
    def _init():
        out_ref[...] = jnp.zeros_like(out_ref)

    out_ref[...] += partial


def _tc_lse_mean(x, feat_t, BT):
    B, D = x.shape
    K = feat_t.shape[1]
    return pl.pallas_call(
        functools.partial(_tc_body, 1.0 / B),
        grid=(B // BT,),
        in_specs=[
            pl.BlockSpec((BT, D), lambda i: (i, 0)),
            pl.BlockSpec((D, K), lambda i: (0, 0)),
        ],
        out_specs=pl.BlockSpec((1, 1), lambda i: (0, 0)),
        out_shape=jax.ShapeDtypeStruct((1, 1), jnp.float32),
    )(x, feat_t)


def kernel(inputs, targets, features):
    B, D = inputs.shape
    K = features.shape[0]
    feat_pad = jnp.pad(features, ((0, 0), (0, 128 - D)))
    t_partials = _sc_target_partials(feat_pad, targets.astype(jnp.int32),
                                     inputs, B, D)
    lse_sum = _tc_lse_mean(inputs, features.T.astype(jnp.bfloat16), 512)
    return lse_sum[0, 0] - jnp.sum(t_partials) * (1.0 / (B * _TEMP))


# speedup vs baseline: 1.0214x; 1.0176x over previous
import functools

import jax
import jax.numpy as jnp
from jax import lax
from jax.experimental import pallas as pl
from jax.experimental.pallas import tpu as pltpu
from jax.experimental.pallas import tpu_sc as plsc

_TEMP = 0.05
_EPS = 1e-12
_LOG2E = 1.4426950408889634


def _sc_target_partials(table_pad, idx, x, B, D):
    info = plsc.get_sparse_core_info()
    nw = info.num_cores * info.num_subcores
    bw = B // nw
    ng = bw // 16
    dpad = table_pad.shape[1]
    mesh = plsc.VectorSubcoreMesh(core_axis_name="c", subcore_axis_name="s")

    @functools.partial(
        pl.kernel,
        mesh=mesh,
        out_type=jax.ShapeDtypeStruct((nw, 16), jnp.float32),
        compiler_params=pltpu.CompilerParams(needs_layout_passes=False),
        scratch_types=[
            pltpu.VMEM((bw,), jnp.int32),
            pltpu.VMEM((bw, dpad), jnp.float32),
            pltpu.VMEM((bw, D), jnp.float32),
            pltpu.VMEM((16,), jnp.float32),
            pltpu.SemaphoreType.DMA,
        ],
    )
    def cos_kernel(table_hbm, idx_hbm, x_hbm, out_hbm, idx_v, rows_v, x_v,
                   p_v, sem):
        wid = lax.axis_index("s") * info.num_cores + lax.axis_index("c")
        base = wid * bw
        pltpu.sync_copy(idx_hbm.at[pl.ds(base, bw)], idx_v)
        pltpu.sync_copy(x_hbm.at[pl.ds(base, bw)], x_v)
        pltpu.async_copy(table_hbm.at[idx_v], rows_v, sem).wait()
        lane = jnp.arange(16, dtype=jnp.int32)
        p = jnp.zeros((16,), jnp.float32)
        for g in range(ng):
            row_ids = lane + (16 * g)
            d = jnp.zeros((16,), jnp.float32)
            s = jnp.zeros((16,), jnp.float32)
            for j in range(D):
                col = jnp.full((16,), j, jnp.int32)
                fv = plsc.load_gather(rows_v, [row_ids, col])
                xv = plsc.load_gather(x_v, [row_ids, col])
                d = d + fv * xv
                s = s + xv * xv
            i = plsc.bitcast(s, jnp.int32)
            y = plsc.bitcast(jnp.int32(0x5F3759DF) - (i >> 1), jnp.float32)
            for _ in range(3):
                y = y * (1.5 - 0.5 * s * y * y)
            p = p + d / (s * y + _EPS)
        p_v[...] = p
        pltpu.sync_copy(p_v, out_hbm.at[wid])

    return cos_kernel(table_pad, idx, x)


def _tc_body(inv_b, x_ref, feat_ref, out_ref):
    i = pl.program_id(0)
    x = x_ref[...]
    norm = jnp.sqrt(jnp.sum(x * x, axis=1, keepdims=True))
    xn = x * (_LOG2E / _TEMP) / (norm + _EPS)
    l2 = lax.dot_general(xn.astype(jnp.bfloat16),
                         feat_ref[...].astype(jnp.bfloat16),
                         (((1,), (1,)), ((), ())),
                         preferred_element_type=jnp.float32)
    se = jnp.sum(jnp.exp2(l2), axis=1, keepdims=True)
    lse = jnp.log(se)
    partial = jnp.sum(lse, keepdims=True) * inv_b

    @pl.when(i == 0)
    def _init():
        out_ref[...] = jnp.zeros_like(out_ref)

    out_ref[...] += partial


def _tc_lse_mean(x, feat_bf, BT):
    B, D = x.shape
    K = feat_bf.shape[0]
    return pl.pallas_call(
        functools.partial(_tc_body, 1.0 / B),
        grid=(B // BT,),
        in_specs=[
            pl.BlockSpec((BT, D), lambda i: (i, 0)),
            pl.BlockSpec((K, D), lambda i: (0, 0)),
        ],
        out_specs=pl.BlockSpec((1, 1), lambda i: (0, 0)),
        out_shape=jax.ShapeDtypeStruct((1, 1), jnp.float32),
    )(x, feat_bf)


def kernel(inputs, targets, features):
    B, D = inputs.shape
    K = features.shape[0]
    feat_pad = jnp.pad(features, ((0, 0), (0, 128 - D)))
    t_partials = _sc_target_partials(feat_pad, targets.astype(jnp.int32),
                                     inputs, B, D)
    lse_sum = _tc_lse_mean(inputs, features, 1024)
    return lse_sum[0, 0] - jnp.sum(t_partials) * (1.0 / (B * _TEMP))


# speedup vs baseline: 1.0216x; 1.0002x over previous
import functools

import jax
import jax.numpy as jnp
from jax import lax
from jax.experimental import pallas as pl
from jax.experimental.pallas import tpu as pltpu
from jax.experimental.pallas import tpu_sc as plsc

_TEMP = 0.05
_EPS = 1e-12
_LOG2E = 1.4426950408889634
_LN2 = 0.6931471805599453


def _sc_target_partials(quads, idx, xn, B, D):
    info = plsc.get_sparse_core_info()
    nw = info.num_cores * info.num_subcores
    bw = B // nw
    ng = bw // 16
    mesh = plsc.VectorSubcoreMesh(core_axis_name="c", subcore_axis_name="s")

    @functools.partial(
        pl.kernel,
        mesh=mesh,
        out_type=jax.ShapeDtypeStruct((nw, 16), jnp.float32),
        compiler_params=pltpu.CompilerParams(needs_layout_passes=False),
        scratch_types=[
            pltpu.VMEM((bw,), jnp.int32),
            pltpu.VMEM((bw,), jnp.int32),
            pltpu.VMEM((bw, 128), jnp.float32),
            pltpu.VMEM((bw, D), jnp.float32),
            pltpu.VMEM((16,), jnp.float32),
            pltpu.SemaphoreType.DMA,
        ],
    )
    def tgt_kernel(quads_hbm, idx_hbm, xn_hbm, out_hbm, idx_v, q_v, rows_v,
                   x_v, p_v, sem):
        wid = lax.axis_index("s") * info.num_cores + lax.axis_index("c")
        base = wid * bw
        pltpu.sync_copy(idx_hbm.at[pl.ds(base, bw)], idx_v)
        pltpu.sync_copy(xn_hbm.at[pl.ds(base, bw)], x_v)
        for g in range(ng):
            sl = pl.ds(16 * g, 16)
            q_v[sl] = idx_v[sl] >> 2
        cp = pltpu.async_copy(quads_hbm.at[q_v], rows_v, sem)
        cp.wait()
        lane = jnp.arange(16, dtype=jnp.int32)
        p = jnp.zeros((16,), jnp.float32)
        for g in range(ng):
            row_ids = lane + (16 * g)
            cbase = (idx_v[pl.ds(16 * g, 16)] & 3) * D
            d = jnp.zeros((16,), jnp.float32)
            for j in range(D):
                fv = plsc.load_gather(rows_v, [row_ids, cbase + j])
                xv = plsc.load_gather(x_v, [row_ids,
                                            jnp.full((16,), j, jnp.int32)])
                d = d + fv * xv
            p = p + d
        p_v[...] = p
        pltpu.sync_copy(p_v, out_hbm.at[wid])

    return tgt_kernel(quads, idx, xn)


def _tc_body(inv_b, xn_ref, feat_ref, out_ref):
    i = pl.program_id(0)
    l2 = lax.dot_general(xn_ref[...].astype(jnp.bfloat16), feat_ref[...],
                         (((1,), (1,)), ((), ())),
                         preferred_element_type=jnp.float32)
    se = jnp.sum(jnp.exp2(l2), axis=1, keepdims=True)
    lse = jnp.log(se)
    partial = jnp.sum(lse, keepdims=True) * inv_b

    @pl.when(i == 0)
    def _init():
        out_ref[...] = jnp.zeros_like(out_ref)

    out_ref[...] += partial


def _tc_lse_mean(xn, feat_bf, BT):
    B, D = xn.shape
    K = feat_bf.shape[0]
    return pl.pallas_call(
        functools.partial(_tc_body, 1.0 / B),
        grid=(B // BT,),
        in_specs=[
            pl.BlockSpec((BT, D), lambda i: (i, 0)),
            pl.BlockSpec((K, D), lambda i: (0, 0)),
        ],
        out_specs=pl.BlockSpec((1, 1), lambda i: (0, 0)),
        out_shape=jax.ShapeDtypeStruct((1, 1), jnp.float32),
    )(xn, feat_bf)


def kernel(inputs, targets, features):
    B, D = inputs.shape
    K = features.shape[0]
    norm = jnp.sqrt(jnp.sum(inputs * inputs, axis=1, keepdims=True))
    xn = inputs * (_LOG2E / _TEMP) / (norm + _EPS)
    quads = features.reshape(K // 4, 4 * D)
    t_partials = _sc_target_partials(quads, targets.astype(jnp.int32), xn,
                                     B, D)
    lse_sum = _tc_lse_mean(xn, features.astype(jnp.bfloat16), 1024)
    return lse_sum[0, 0] - jnp.sum(t_partials) * (_LN2 / B)


# speedup vs baseline: 1.0414x; 1.0194x over previous
import functools

import jax
import jax.numpy as jnp
from jax import lax
from jax.experimental import pallas as pl
from jax.experimental.pallas import tpu as pltpu
from jax.experimental.pallas import tpu_sc as plsc

_TEMP = 0.05
_EPS = 1e-12
_LOG2E = 1.4426950408889634


def _sc_target_partials(quads, idx, x, B, D):
    info = plsc.get_sparse_core_info()
    nw = info.num_cores * info.num_subcores
    bw = B // nw
    ng = bw // 16
    mesh = plsc.VectorSubcoreMesh(core_axis_name="c", subcore_axis_name="s")

    @functools.partial(
        pl.kernel,
        mesh=mesh,
        out_type=jax.ShapeDtypeStruct((nw, 16), jnp.float32),
        compiler_params=pltpu.CompilerParams(needs_layout_passes=False),
        scratch_types=[
            pltpu.VMEM((bw,), jnp.int32),
            pltpu.VMEM((bw,), jnp.int32),
            pltpu.VMEM((bw, 4 * D), jnp.float32),
            pltpu.VMEM((bw, D), jnp.float32),
            pltpu.VMEM((16,), jnp.float32),
            pltpu.SemaphoreType.DMA,
        ],
    )
    def tgt_kernel(quads_hbm, idx_hbm, x_hbm, out_hbm, idx_v, q_v, rows_v,
                   x_v, p_v, sem):
        wid = lax.axis_index("s") * info.num_cores + lax.axis_index("c")
        base = wid * bw
        pltpu.sync_copy(idx_hbm.at[pl.ds(base, bw)], idx_v)
        pltpu.sync_copy(x_hbm.at[pl.ds(base, bw)], x_v)
        for g in range(ng):
            sl = pl.ds(16 * g, 16)
            q_v[sl] = idx_v[sl] >> 2
        cp = pltpu.async_copy(quads_hbm.at[q_v], rows_v, sem)
        cp.wait()
        lane = jnp.arange(16, dtype=jnp.int32)
        p = jnp.zeros((16,), jnp.float32)
        for g in range(ng):
            row_ids = lane + (16 * g)
            cbase = (idx_v[pl.ds(16 * g, 16)] & 3) * D
            d = jnp.zeros((16,), jnp.float32)
            s = jnp.zeros((16,), jnp.float32)
            for j in range(D):
                fv = plsc.load_gather(rows_v, [row_ids, cbase + j])
                xv = plsc.load_gather(x_v, [row_ids,
                                            jnp.full((16,), j, jnp.int32)])
                d = d + fv * xv
                s = s + xv * xv
            i = plsc.bitcast(s, jnp.int32)
            y = plsc.bitcast(jnp.int32(0x5F3759DF) - (i >> 1), jnp.float32)
            for _ in range(3):
                y = y * (1.5 - 0.5 * s * y * y)
            p = p + d / (s * y + _EPS)
        p_v[...] = p
        pltpu.sync_copy(p_v, out_hbm.at[wid])

    return tgt_kernel(quads, idx, x)


def _tc_body(inv_b, x_ref, feat_ref, out_ref):
    i = pl.program_id(0)
    x = x_ref[...]
    norm = jnp.sqrt(jnp.sum(x * x, axis=1, keepdims=True))
    xn = x * (_LOG2E / _TEMP) / (norm + _EPS)
    l2 = lax.dot_general(xn.astype(jnp.bfloat16),
                         feat_ref[...].astype(jnp.bfloat16),
                         (((1,), (1,)), ((), ())),
                         preferred_element_type=jnp.float32)
    se = jnp.sum(jnp.exp2(l2), axis=1, keepdims=True)
    lse = jnp.log(se)
    partial = jnp.sum(lse, keepdims=True) * inv_b

    @pl.when(i == 0)
    def _init():
        out_ref[...] = jnp.zeros_like(out_ref)

    out_ref[...] += partial


def _tc_lse_mean(x, feat, BT):
    B, D = x.shape
    K = feat.shape[0]
    return pl.pallas_call(
        functools.partial(_tc_body, 1.0 / B),
        grid=(B // BT,),
        in_specs=[
            pl.BlockSpec((BT, D), lambda i: (i, 0)),
            pl.BlockSpec((K, D), lambda i: (0, 0)),
        ],
        out_specs=pl.BlockSpec((1, 1), lambda i: (0, 0)),
        out_shape=jax.ShapeDtypeStruct((1, 1), jnp.float32),
    )(x, feat)


def kernel(inputs, targets, features):
    B, D = inputs.shape
    K = features.shape[0]
    quads = features.reshape(K // 4, 4 * D)
    t_partials = _sc_target_partials(quads, targets.astype(jnp.int32),
                                     inputs, B, D)
    lse_sum = _tc_lse_mean(inputs, features, 1024)
    return lse_sum[0, 0] - jnp.sum(t_partials) * (1.0 / (B * _TEMP))
